# baseline (speedup 1.0000x reference)
import jax
import jax.numpy as jnp
from jax.experimental import pallas as pl
from jax.experimental.pallas import tpu as pltpu

BINS = 25
KP = 5
EPS = 1e-5
F32 = jnp.float32


def _entropy(y):
    ho = y.shape[0] - KP + 1
    wo = y.shape[1] - KP + 1
    slabs = [y[i:i + ho, j:j + wo, :] for i in range(KP) for j in range(KP)]
    mn = slabs[0]
    mx = slabs[0]
    for s in slabs[1:]:
        mn = jnp.minimum(mn, s)
        mx = jnp.maximum(mx, s)
    rng = jnp.where(mx > mn, mx - mn, 1.0)
    inv = BINS / rng
    qs = [jnp.clip(jnp.floor((s - mn) * inv), 0.0, BINS - 1.0) for s in slabs]
    q = jnp.stack(qs)
    ent = jnp.zeros(mn.shape, F32)
    for k in range(BINS):
        pk = (q == float(k)).astype(F32).sum(axis=0) * (1.0 / (KP * KP))
        ent = ent - pk * jnp.log(jnp.clip(pk, 1e-5, 1.0 - 1e-5))
    return ent.mean(axis=(0, 1))


def _k1(xp_ref, w_ref, cb_ref, gw_ref, gb_ref, o_ref):
    acc = jnp.zeros((8, 56, 56), F32)
    for ky in range(7):
        dy, py = divmod(ky, 4)
        for kx in range(7):
            dx, px = divmod(kx, 4)
            for c in range(3):
                slab = xp_ref[0, py, px, c, dy:dy + 56, dx:dx + 56]
                acc = acc + w_ref[:, c, ky, kx][:, None, None] * slab[None, :, :]
    acc = acc + cb_ref[...]
    mu = acc.mean(axis=(1, 2), keepdims=True)
    d = acc - mu
    v = (d * d).mean(axis=(1, 2), keepdims=True)
    y = d * jax.lax.rsqrt(v + EPS) * gw_ref[...] + gb_ref[...]
    o_ref[0] = jnp.maximum(y, 0.0)


def _make_mid(stride, ks, cin, ho):
    n = float(ho * ho)

    def body(xp_ref, w_ref, cb_ref, nw_ref, nb_ref, o_ref):
        acc = None
        for ky in range(ks):
            dy, py = divmod(ky, stride)
            for kx in range(ks):
                dx, px = divmod(kx, stride)
                for c in range(cin):
                    t = w_ref[0, c, ky, kx] * xp_ref[py, px, c, dy:dy + ho, dx:dx + ho, :]
                    acc = t if acc is None else acc + t
        acc = acc + cb_ref[0, 0, 0]
        ent = _entropy(acc)
        mu = acc.mean(axis=(0, 1))
        d = acc - mu
        v = (d * d).mean(axis=(0, 1)) * (n / (n - 1.0))
        y = d * jax.lax.rsqrt(v + EPS) * (nw_ref[0, 0, 0] * ent) + nb_ref[0, 0, 0]
        o_ref[0] = jnp.maximum(y, 0.0)

    return body


_k2 = _make_mid(3, 5, 8, 18)
_k3 = _make_mid(1, 3, 16, 18)


def _k4(h_ref, g_ref, b_ref, fw_ref, fb_ref, sw_ref, sb_ref, vw_ref, vb_ref,
        os_ref, ov_ref):
    h = h_ref[...]
    mu = h.mean(axis=0, keepdims=True)
    d = h - mu
    v = (d * d).mean(axis=0, keepdims=True)
    hn = d * jax.lax.rsqrt(v + EPS) * g_ref[...] + b_ref[...]
    f = jax.lax.dot_general(hn, fw_ref[...], (((1,), (1,)), ((), ())),
                            preferred_element_type=F32)
    f = jnp.maximum(f + fb_ref[...], 0.0)

    def head(wv, bv):
        l = jax.lax.dot_general(f, wv, (((1,), (1,)), ((), ())),
                                preferred_element_type=F32) + bv
        m = l.max(axis=1, keepdims=True)
        e = jnp.exp(l - m)
        return e / e.sum(axis=1, keepdims=True)

    os_ref[...] = head(sw_ref[...], sb_ref[...])
    ov_ref[...] = head(vw_ref[...], vb_ref[...])


def _cparams():
    return pltpu.CompilerParams(dimension_semantics=("parallel",),
                                vmem_limit_bytes=56 * 1024 * 1024)


def kernel(x, conv0_w, conv0_b, conv1_w, conv1_b, conv2_w, conv2_b,
           gn0_w, gn0_b, n1_w, n1_b, n2_w, n2_b,
           bn_g, bn_b, fc1_w, fc1_b, shape_w, shape_b, vern_w, vern_b):
    b = x.shape[0]

    xp = jnp.pad(x, ((0, 0), (0, 0), (2, 2), (2, 2)))
    xp = xp.reshape(b, 3, 57, 4, 57, 4).transpose(0, 3, 5, 1, 2, 4)
    h0 = pl.pallas_call(
        _k1,
        grid=(b,),
        in_specs=[
            pl.BlockSpec((1, 4, 4, 3, 57, 57), lambda i: (i, 0, 0, 0, 0, 0)),
            pl.BlockSpec((8, 3, 7, 7), lambda i: (0, 0, 0, 0)),
            pl.BlockSpec((8, 1, 1), lambda i: (0, 0, 0)),
            pl.BlockSpec((8, 1, 1), lambda i: (0, 0, 0)),
            pl.BlockSpec((8, 1, 1), lambda i: (0, 0, 0)),
        ],
        out_specs=pl.BlockSpec((1, 8, 56, 56), lambda i: (i, 0, 0, 0)),
        out_shape=jax.ShapeDtypeStruct((b, 8, 56, 56), F32),
        compiler_params=_cparams(),
        name="conv0_gn",
    )(xp, conv0_w, conv0_b.reshape(8, 1, 1), gn0_w.reshape(8, 1, 1),
      gn0_b.reshape(8, 1, 1))

    h0p = jnp.pad(h0, ((0, 0), (0, 0), (1, 3), (1, 3)))
    h0p = h0p.reshape(b, 8, 20, 3, 20, 3).transpose(3, 5, 1, 2, 4, 0)
    h1 = pl.pallas_call(
        _k2,
        grid=(16,),
        in_specs=[
            pl.BlockSpec((3, 3, 8, 20, 20, b), lambda i: (0, 0, 0, 0, 0, 0)),
            pl.BlockSpec((1, 8, 5, 5), lambda i: (i, 0, 0, 0)),
            pl.BlockSpec((1, 1, 1), lambda i: (i, 0, 0)),
            pl.BlockSpec((1, 1, 1), lambda i: (i, 0, 0)),
            pl.BlockSpec((1, 1, 1), lambda i: (i, 0, 0)),
        ],
        out_specs=pl.BlockSpec((1, 18, 18, b), lambda i: (i, 0, 0, 0)),
        out_shape=jax.ShapeDtypeStruct((16, 18, 18, b), F32),
        compiler_params=_cparams(),
        name="conv1_ent",
    )(h0p, conv1_w, conv1_b.reshape(16, 1, 1), n1_w.reshape(16, 1, 1),
      n1_b.reshape(16, 1, 1))

    h1p = jnp.pad(h1, ((0, 0), (1, 1), (1, 1), (0, 0)))
    h1p = h1p.reshape(1, 1, 16, 20, 20, b)
    h2 = pl.pallas_call(
        _k3,
        grid=(32,),
        in_specs=[
            pl.BlockSpec((1, 1, 16, 20, 20, b), lambda i: (0, 0, 0, 0, 0, 0)),
            pl.BlockSpec((1, 16, 3, 3), lambda i: (i, 0, 0, 0)),
            pl.BlockSpec((1, 1, 1), lambda i: (i, 0, 0)),
            pl.BlockSpec((1, 1, 1), lambda i: (i, 0, 0)),
            pl.BlockSpec((1, 1, 1), lambda i: (i, 0, 0)),
        ],
        out_specs=pl.BlockSpec((1, 18, 18, b), lambda i: (i, 0, 0, 0)),
        out_shape=jax.ShapeDtypeStruct((32, 18, 18, b), F32),
        compiler_params=_cparams(),
        name="conv2_ent",
    )(h1p, conv2_w, conv2_b.reshape(32, 1, 1), n2_w.reshape(32, 1, 1),
      n2_b.reshape(32, 1, 1))

    hf = h2.transpose(3, 0, 1, 2).reshape(b, 32 * 18 * 18)
    out_s, out_v = pl.pallas_call(
        _k4,
        out_shape=(jax.ShapeDtypeStruct((b, 5), F32),
                   jax.ShapeDtypeStruct((b, 2), F32)),
        name="bn_fc_heads",
    )(hf, bn_g.reshape(1, -1), bn_b.reshape(1, -1), fc1_w,
      fc1_b.reshape(1, -1), shape_w, shape_b.reshape(1, -1), vern_w,
      vern_b.reshape(1, -1))
    return (out_s, out_v)
